# Initial kernel scaffold; baseline (speedup 1.0000x reference)
import jax
import jax.numpy as jnp
from jax.experimental import pallas as pl


def kernel(x, edge_index, Ws0, Wn0, b0, Ws1, Wn1, b1, Ws2, Wn2, b2):
    raise NotImplementedError("write your pallas kernel here")


# speedup vs baseline: 2.8449x; 2.8449x over previous
import functools

import jax
import jax.numpy as jnp
from jax import lax
from jax.experimental import pallas as pl
from jax.experimental.pallas import tpu as pltpu
from jax.experimental.pallas import tpu_sc as plsc

N = 10000
E = 160000
CH = 64
EPAD = 163840
EROWS = EPAD // 128
NC, NS = 2, 16
NPAD = 10016
SLAB = 624
TAIL_O = N - 15 * SLAB
TAIL_Z = NPAD - 15 * SLAB
G = 4
RPS = EROWS // NS

_MESH = plsc.VectorSubcoreMesh(core_axis_name="c", subcore_axis_name="s")


def _zero_acc(sub, zeros, acc):
    @pl.when(sub < NS - 1)
    def _():
        pltpu.sync_copy(zeros.at[pl.ds(0, SLAB)],
                        acc.at[pl.ds(sub * SLAB, SLAB)])
    @pl.when(sub == NS - 1)
    def _():
        pltpu.sync_copy(zeros, acc.at[pl.ds(15 * SLAB, TAIL_Z)])


def _copy_out(sub, acc, out, off):
    @pl.when(sub < NS - 1)
    def _():
        pltpu.sync_copy(acc.at[pl.ds(sub * SLAB, SLAB)],
                        out.at[pl.ds(off + sub * SLAB, SLAB)])
    @pl.when(sub == NS - 1)
    def _():
        pltpu.sync_copy(acc.at[pl.ds(15 * SLAB, TAIL_O)],
                        out.at[pl.ds(off + 15 * SLAB, TAIL_O)])


def _make_segsum(P, with_deg):
    n_out = P * NC * N + (N if with_deg else 0)

    @functools.partial(
        pl.kernel,
        out_type=jax.ShapeDtypeStruct((n_out, CH), jnp.float32),
        mesh=_MESH,
        compiler_params=pltpu.CompilerParams(use_tc_tiling_on_sc=False),
        scratch_types=[
            pltpu.VMEM((RPS, 128), jnp.int32),
            pltpu.VMEM((RPS, 128), jnp.int32),
            pltpu.VMEM((G, 128, CH), jnp.float32),
            pltpu.VMEM((128, CH), jnp.float32),
            pltpu.VMEM_SHARED((NPAD, CH), jnp.float32),
            pltpu.SemaphoreType.DMA,
        ],
    )
    def segsum(h_t, src2, dst2, zeros, ones, out,
               idx_s, idx_d, rows, ones_v, acc, sem):
        core = lax.axis_index("c")
        sub = lax.axis_index("s")
        row0 = sub * RPS
        pltpu.sync_copy(src2.at[pl.ds(row0, RPS)], idx_s)
        pltpu.sync_copy(dst2.at[pl.ds(row0, RPS)], idx_d)

        def shift(delta):
            def body(i, _):
                r = i // 8
                c = (i % 8) * 16
                idx_s[r, pl.ds(c, 16)] = idx_s[r, pl.ds(c, 16)] + delta
                return 0
            lax.fori_loop(0, RPS * 8, body, 0)

        for p in range(P):
            shift(core * (P * N) if p == 0 else N)
            _zero_acc(sub, zeros, acc)
            plsc.subcore_barrier()

            def group(g, _):
                base = g * G
                cps = [
                    pltpu.async_copy(h_t.at[idx_s.at[base + j]], rows.at[j], sem)
                    for j in range(G)
                ]
                for cp in cps:
                    cp.wait()
                for j in range(G):
                    pltpu.sync_copy(rows.at[j], acc.at[idx_d.at[base + j]],
                                    add=True)
                return 0

            lax.fori_loop(0, RPS // G, group, 0)
            plsc.subcore_barrier()
            _copy_out(sub, acc, out, (core * P + p) * N)

        if with_deg:
            plsc.subcore_barrier()
            pltpu.sync_copy(ones, ones_v)
            _zero_acc(sub, zeros, acc)
            plsc.subcore_barrier()

            def deg_body(r, _):
                pltpu.sync_copy(ones_v, acc.at[idx_d.at[r]], add=True)
                return 0

            lax.fori_loop(0, RPS, deg_body, 0)
            plsc.subcore_barrier()
            @pl.when(core == 0)
            def _():
                _copy_out(sub, acc, out, P * NC * N)

    return segsum


_segsum_w256_deg = _make_segsum(2, True)
_segsum_w512 = _make_segsum(4, False)
_segsum_w256 = _make_segsum(2, False)

_TC_R = 2000


def _layer_body(h_ref, agg_ref, deg_ref, ws_ref, wn_ref, b_ref, out_ref):
    inv = 1.0 / jnp.maximum(deg_ref[...], 1.0)
    mean = agg_ref[...] * inv
    acc = jnp.dot(h_ref[...], ws_ref[...], preferred_element_type=jnp.float32)
    acc = acc + jnp.dot(mean, wn_ref[...], preferred_element_type=jnp.float32)
    out_ref[...] = jnp.maximum(acc + b_ref[...], 0.0)


def _tc_layer(h, agg, deg, Ws, Wn, b):
    fin, fout = Ws.shape
    return pl.pallas_call(
        _layer_body,
        grid=(N // _TC_R,),
        in_specs=[
            pl.BlockSpec((_TC_R, fin), lambda i: (i, 0)),
            pl.BlockSpec((_TC_R, fin), lambda i: (i, 0)),
            pl.BlockSpec((_TC_R, 1), lambda i: (i, 0)),
            pl.BlockSpec((fin, fout), lambda i: (0, 0)),
            pl.BlockSpec((fin, fout), lambda i: (0, 0)),
            pl.BlockSpec((1, fout), lambda i: (0, 0)),
        ],
        out_specs=pl.BlockSpec((_TC_R, fout), lambda i: (i, 0)),
        out_shape=jax.ShapeDtypeStruct((N, fout), jnp.float32),
    )(h, agg, deg, Ws, Wn, b.reshape(1, fout))


def _proj_body(h_ref, w_ref, out_ref):
    out_ref[...] = jnp.dot(h_ref[...], w_ref[...],
                           preferred_element_type=jnp.float32)


def _tc_proj(h, W):
    fin, fout = W.shape
    return pl.pallas_call(
        _proj_body,
        grid=(N // _TC_R,),
        in_specs=[
            pl.BlockSpec((_TC_R, fin), lambda i: (i, 0)),
            pl.BlockSpec((fin, fout), lambda i: (0, 0)),
        ],
        out_specs=pl.BlockSpec((_TC_R, fout), lambda i: (i, 0)),
        out_shape=jax.ShapeDtypeStruct((N, fout), jnp.float32),
    )(h, W)


def _final_body(h_ref, agg_ref, deg_ref, ws_ref, b_ref, out_ref):
    inv = 1.0 / jnp.maximum(deg_ref[...], 1.0)
    acc = jnp.dot(h_ref[...], ws_ref[...], preferred_element_type=jnp.float32)
    out_ref[...] = jnp.maximum(acc + agg_ref[...] * inv + b_ref[...], 0.0)


def _tc_final(h, agg, deg, Ws, b):
    fin, fout = Ws.shape
    return pl.pallas_call(
        _final_body,
        grid=(N // _TC_R,),
        in_specs=[
            pl.BlockSpec((_TC_R, fin), lambda i: (i, 0)),
            pl.BlockSpec((_TC_R, fout), lambda i: (i, 0)),
            pl.BlockSpec((_TC_R, 1), lambda i: (i, 0)),
            pl.BlockSpec((fin, fout), lambda i: (0, 0)),
            pl.BlockSpec((1, fout), lambda i: (0, 0)),
        ],
        out_specs=pl.BlockSpec((_TC_R, fout), lambda i: (i, 0)),
        out_shape=jax.ShapeDtypeStruct((N, fout), jnp.float32),
    )(h, agg, deg, Ws, b.reshape(1, fout))


def _to_chunks(h, P):
    return h.reshape(N, P * NC, CH).transpose(1, 0, 2).reshape(P * NC * N, CH)


def _from_chunks(a, P):
    return a.reshape(P * NC, N, CH).transpose(1, 0, 2).reshape(N, P * NC * CH)


def kernel(x, edge_index, Ws0, Wn0, b0, Ws1, Wn1, b1, Ws2, Wn2, b2):
    src = edge_index[0].astype(jnp.int32)
    dst = edge_index[1].astype(jnp.int32)
    pad = EPAD - E
    src2 = jnp.concatenate([src, jnp.zeros((pad,), jnp.int32)]).reshape(EROWS, 128)
    dst2 = jnp.concatenate([dst, jnp.full((pad,), N, jnp.int32)]).reshape(EROWS, 128)

    zeros = jnp.zeros((TAIL_Z, CH), jnp.float32)
    ones = jnp.ones((128, CH), jnp.float32)

    out0 = _segsum_w256_deg(_to_chunks(x, 2), src2, dst2, zeros, ones)
    agg0 = _from_chunks(out0[: 2 * NC * N], 2)
    deg = out0[2 * NC * N :, :1]
    h1 = _tc_layer(x, agg0, deg, Ws0, Wn0, b0)

    agg1 = _from_chunks(
        _segsum_w512(_to_chunks(h1, 4), src2, dst2, zeros, ones), 4)
    h2 = _tc_layer(h1, agg1, deg, Ws1, Wn1, b1)

    hp = _tc_proj(h2, Wn2)
    agg2 = _from_chunks(
        _segsum_w256(_to_chunks(hp, 2), src2, dst2, zeros, ones), 2)
    return _tc_final(h2, agg2, deg, Ws2, b2)


# speedup vs baseline: 3.0280x; 1.0644x over previous
import functools

import jax
import jax.numpy as jnp
from jax import lax
from jax.experimental import pallas as pl
from jax.experimental.pallas import tpu as pltpu
from jax.experimental.pallas import tpu_sc as plsc

N = 10000
E = 160000
CH = 64
EPAD = 163840
EROWS = EPAD // 128
NC, NS = 2, 16
NPAD = 10016
SLAB = 624
TAIL_O = N - 15 * SLAB
TAIL_Z = NPAD - 15 * SLAB
G = 2
RPS = EROWS // NS

_MESH = plsc.VectorSubcoreMesh(core_axis_name="c", subcore_axis_name="s")


def _zero_acc(sub, zeros, acc):
    @pl.when(sub < NS - 1)
    def _():
        pltpu.sync_copy(zeros.at[pl.ds(0, SLAB)],
                        acc.at[pl.ds(sub * SLAB, SLAB)])
    @pl.when(sub == NS - 1)
    def _():
        pltpu.sync_copy(zeros, acc.at[pl.ds(15 * SLAB, TAIL_Z)])


def _copy_out(sub, acc, out, off):
    @pl.when(sub < NS - 1)
    def _():
        pltpu.sync_copy(acc.at[pl.ds(sub * SLAB, SLAB)],
                        out.at[pl.ds(off + sub * SLAB, SLAB)])
    @pl.when(sub == NS - 1)
    def _():
        pltpu.sync_copy(acc.at[pl.ds(15 * SLAB, TAIL_O)],
                        out.at[pl.ds(off + 15 * SLAB, TAIL_O)])


def _make_segsum(P, with_deg):
    n_out = P * NC * N + (N if with_deg else 0)

    @functools.partial(
        pl.kernel,
        out_type=jax.ShapeDtypeStruct((n_out, CH), jnp.float32),
        mesh=_MESH,
        compiler_params=pltpu.CompilerParams(use_tc_tiling_on_sc=False),
        scratch_types=[
            pltpu.VMEM((RPS, 128), jnp.int32),
            pltpu.VMEM((RPS, 128), jnp.int32),
            pltpu.VMEM((2, G, 128, CH), jnp.float32),
            pltpu.VMEM_SHARED((NPAD, CH), jnp.float32),
            pltpu.SemaphoreType.DMA,
            pltpu.SemaphoreType.DMA,
            pltpu.SemaphoreType.DMA,
        ],
    )
    def segsum(h_t, src2, dst2, zeros, ones, out,
               idx_s, idx_d, rows, acc, sem_g, sem_s0, sem_s1):
        core = lax.axis_index("c")
        sub = lax.axis_index("s")
        row0 = sub * RPS
        pltpu.sync_copy(src2.at[pl.ds(row0, RPS)], idx_s)
        pltpu.sync_copy(dst2.at[pl.ds(row0, RPS)], idx_d)

        def shift(delta):
            def body(i, _):
                r = i // 8
                c = (i % 8) * 16
                idx_s[r, pl.ds(c, 16)] = idx_s[r, pl.ds(c, 16)] + delta
                return 0
            lax.fori_loop(0, RPS * 8, body, 0)

        NG = RPS // G

        def fire_g(grp, buf):
            for j in range(G):
                pltpu.async_copy(h_t.at[idx_s.at[grp * G + j]],
                                 rows.at[buf, j], sem_g)

        def wait_g(buf):
            for j in range(G):
                pltpu.make_async_copy(h_t.at[idx_s.at[0]],
                                      rows.at[buf, j], sem_g).wait()

        def fire_s(grp, buf, sem):
            for j in range(G):
                pltpu.async_copy(rows.at[buf, j],
                                 acc.at[idx_d.at[grp * G + j]], sem, add=True)

        def wait_s(buf, sem):
            for j in range(G):
                pltpu.make_async_copy(rows.at[buf, j],
                                      acc.at[idx_d.at[0]], sem).wait()

        for p in range(P):
            shift(core * (P * N) if p == 0 else N)
            _zero_acc(sub, zeros, acc)
            plsc.subcore_barrier()

            fire_g(0, 0)

            def pipe(k2, _):
                wait_g(0)
                fire_s(2 * k2, 0, sem_s0)
                @pl.when(k2 > 0)
                def _():
                    wait_s(1, sem_s1)
                fire_g(2 * k2 + 1, 1)
                wait_g(1)
                fire_s(2 * k2 + 1, 1, sem_s1)
                wait_s(0, sem_s0)
                @pl.when(k2 < NG // 2 - 1)
                def _():
                    fire_g(2 * k2 + 2, 0)
                return 0

            lax.fori_loop(0, NG // 2, pipe, 0)
            wait_s(1, sem_s1)
            plsc.subcore_barrier()
            _copy_out(sub, acc, out, (core * P + p) * N)

        if with_deg:
            plsc.subcore_barrier()
            pltpu.sync_copy(ones, rows.at[0, 0])
            _zero_acc(sub, zeros, acc)
            plsc.subcore_barrier()

            def deg_body(r, _):
                pltpu.sync_copy(rows.at[0, 0], acc.at[idx_d.at[r]], add=True)
                return 0

            lax.fori_loop(0, RPS, deg_body, 0)
            plsc.subcore_barrier()
            @pl.when(core == 0)
            def _():
                _copy_out(sub, acc, out, P * NC * N)

    return segsum


_segsum_w256_deg = _make_segsum(2, True)
_segsum_w512 = _make_segsum(4, False)
_segsum_w256 = _make_segsum(2, False)

_TC_R = 2000


def _layer_body(h_ref, agg_ref, deg_ref, ws_ref, wn_ref, b_ref, out_ref):
    inv = 1.0 / jnp.maximum(deg_ref[...], 1.0)
    mean = agg_ref[...] * inv
    acc = jnp.dot(h_ref[...], ws_ref[...], preferred_element_type=jnp.float32)
    acc = acc + jnp.dot(mean, wn_ref[...], preferred_element_type=jnp.float32)
    out_ref[...] = jnp.maximum(acc + b_ref[...], 0.0)


def _tc_layer(h, agg, deg, Ws, Wn, b):
    fin, fout = Ws.shape
    return pl.pallas_call(
        _layer_body,
        grid=(N // _TC_R,),
        in_specs=[
            pl.BlockSpec((_TC_R, fin), lambda i: (i, 0)),
            pl.BlockSpec((_TC_R, fin), lambda i: (i, 0)),
            pl.BlockSpec((_TC_R, 1), lambda i: (i, 0)),
            pl.BlockSpec((fin, fout), lambda i: (0, 0)),
            pl.BlockSpec((fin, fout), lambda i: (0, 0)),
            pl.BlockSpec((1, fout), lambda i: (0, 0)),
        ],
        out_specs=pl.BlockSpec((_TC_R, fout), lambda i: (i, 0)),
        out_shape=jax.ShapeDtypeStruct((N, fout), jnp.float32),
    )(h, agg, deg, Ws, Wn, b.reshape(1, fout))


def _proj_body(h_ref, w_ref, out_ref):
    out_ref[...] = jnp.dot(h_ref[...], w_ref[...],
                           preferred_element_type=jnp.float32)


def _tc_proj(h, W):
    fin, fout = W.shape
    return pl.pallas_call(
        _proj_body,
        grid=(N // _TC_R,),
        in_specs=[
            pl.BlockSpec((_TC_R, fin), lambda i: (i, 0)),
            pl.BlockSpec((fin, fout), lambda i: (0, 0)),
        ],
        out_specs=pl.BlockSpec((_TC_R, fout), lambda i: (i, 0)),
        out_shape=jax.ShapeDtypeStruct((N, fout), jnp.float32),
    )(h, W)


def _final_body(h_ref, agg_ref, deg_ref, ws_ref, b_ref, out_ref):
    inv = 1.0 / jnp.maximum(deg_ref[...], 1.0)
    acc = jnp.dot(h_ref[...], ws_ref[...], preferred_element_type=jnp.float32)
    out_ref[...] = jnp.maximum(acc + agg_ref[...] * inv + b_ref[...], 0.0)


def _tc_final(h, agg, deg, Ws, b):
    fin, fout = Ws.shape
    return pl.pallas_call(
        _final_body,
        grid=(N // _TC_R,),
        in_specs=[
            pl.BlockSpec((_TC_R, fin), lambda i: (i, 0)),
            pl.BlockSpec((_TC_R, fout), lambda i: (i, 0)),
            pl.BlockSpec((_TC_R, 1), lambda i: (i, 0)),
            pl.BlockSpec((fin, fout), lambda i: (0, 0)),
            pl.BlockSpec((1, fout), lambda i: (0, 0)),
        ],
        out_specs=pl.BlockSpec((_TC_R, fout), lambda i: (i, 0)),
        out_shape=jax.ShapeDtypeStruct((N, fout), jnp.float32),
    )(h, agg, deg, Ws, b.reshape(1, fout))


def _to_chunks(h, P):
    return h.reshape(N, P * NC, CH).transpose(1, 0, 2).reshape(P * NC * N, CH)


def _from_chunks(a, P):
    return a.reshape(P * NC, N, CH).transpose(1, 0, 2).reshape(N, P * NC * CH)


def kernel(x, edge_index, Ws0, Wn0, b0, Ws1, Wn1, b1, Ws2, Wn2, b2):
    src = edge_index[0].astype(jnp.int32)
    dst = edge_index[1].astype(jnp.int32)
    pad = EPAD - E
    src2 = jnp.concatenate([src, jnp.zeros((pad,), jnp.int32)]).reshape(EROWS, 128)
    dst2 = jnp.concatenate([dst, jnp.full((pad,), N, jnp.int32)]).reshape(EROWS, 128)

    zeros = jnp.zeros((TAIL_Z, CH), jnp.float32)
    ones = jnp.ones((128, CH), jnp.float32)

    out0 = _segsum_w256_deg(_to_chunks(x, 2), src2, dst2, zeros, ones)
    agg0 = _from_chunks(out0[: 2 * NC * N], 2)
    deg = out0[2 * NC * N :, :1]
    h1 = _tc_layer(x, agg0, deg, Ws0, Wn0, b0)

    agg1 = _from_chunks(
        _segsum_w512(_to_chunks(h1, 4), src2, dst2, zeros, ones), 4)
    h2 = _tc_layer(h1, agg1, deg, Ws1, Wn1, b1)

    hp = _tc_proj(h2, Wn2)
    agg2 = _from_chunks(
        _segsum_w256(_to_chunks(hp, 2), src2, dst2, zeros, ones), 2)
    return _tc_final(h2, agg2, deg, Ws2, b2)


# speedup vs baseline: 3.0413x; 1.0044x over previous
import functools

import jax
import jax.numpy as jnp
from jax import lax
from jax.experimental import pallas as pl
from jax.experimental.pallas import tpu as pltpu
from jax.experimental.pallas import tpu_sc as plsc

N = 10000
E = 160000
CH = 64
EPAD = 163840
EROWS = EPAD // 128
NC, NS = 2, 16
NPAD = 10016
SLAB = 624
TAIL_O = N - 15 * SLAB
TAIL_Z = NPAD - 15 * SLAB
IR = 256
NR = EPAD // IR // NS
SRO = 128
NSR = EPAD // SRO // NS

_MESH = plsc.VectorSubcoreMesh(core_axis_name="c", subcore_axis_name="s")


def _zero_acc(sub, zeros, acc):
    @pl.when(sub < NS - 1)
    def _():
        pltpu.sync_copy(zeros.at[pl.ds(0, SLAB)],
                        acc.at[pl.ds(sub * SLAB, SLAB)])
    @pl.when(sub == NS - 1)
    def _():
        pltpu.sync_copy(zeros, acc.at[pl.ds(15 * SLAB, TAIL_Z)])


def _copy_out(sub, acc, out, off):
    @pl.when(sub < NS - 1)
    def _():
        pltpu.sync_copy(acc.at[pl.ds(sub * SLAB, SLAB)],
                        out.at[pl.ds(off + sub * SLAB, SLAB)])
    @pl.when(sub == NS - 1)
    def _():
        pltpu.sync_copy(acc.at[pl.ds(15 * SLAB, TAIL_O)],
                        out.at[pl.ds(off + 15 * SLAB, TAIL_O)])


def _make_segsum(P, with_deg):
    n_out = P * NC * N + (N if with_deg else 0)

    @functools.partial(
        pl.kernel,
        out_type=jax.ShapeDtypeStruct((n_out, CH), jnp.float32),
        mesh=_MESH,
        compiler_params=pltpu.CompilerParams(use_tc_tiling_on_sc=False),
        scratch_types=[
            pltpu.VMEM((NR * IR,), jnp.int32),
            pltpu.VMEM((NSR, SRO), jnp.int32),
            pltpu.VMEM((2, IR, CH), jnp.float32),
            pltpu.VMEM_SHARED((NPAD, CH), jnp.float32),
            pltpu.SemaphoreType.DMA,
            pltpu.SemaphoreType.DMA,
            pltpu.SemaphoreType.DMA,
        ],
    )
    def segsum(h_t, src1, dst2, zeros, ones, out,
               idx_s, idx_d, rows, acc, sem_g, sem_s0, sem_s1):
        core = lax.axis_index("c")
        sub = lax.axis_index("s")
        pltpu.sync_copy(src1.at[pl.ds(sub * NR * IR, NR * IR)], idx_s)
        pltpu.sync_copy(dst2.at[pl.ds(sub * NSR, NSR)], idx_d)

        def shift(delta):
            def body(i, _):
                idx_s[pl.ds(i * 16, 16)] = idx_s[pl.ds(i * 16, 16)] + delta
                return 0
            lax.fori_loop(0, NR * IR // 16, body, 0)

        NG = NR

        def fire_g(grp, buf):
            pltpu.async_copy(h_t.at[idx_s.at[pl.ds(grp * IR, IR)]],
                             rows.at[buf], sem_g)

        def wait_g(buf):
            pltpu.make_async_copy(h_t.at[idx_s.at[pl.ds(0, IR)]],
                                  rows.at[buf], sem_g).wait()

        def fire_s(grp, buf, sem):
            for j in range(IR // SRO):
                pltpu.async_copy(rows.at[buf, pl.ds(j * SRO, SRO)],
                                 acc.at[idx_d.at[grp * (IR // SRO) + j]],
                                 sem, add=True)

        def wait_s(buf, sem):
            for j in range(IR // SRO):
                pltpu.make_async_copy(rows.at[buf, pl.ds(j * SRO, SRO)],
                                      acc.at[idx_d.at[0]], sem).wait()

        for p in range(P):
            shift(core * (P * N) if p == 0 else N)
            _zero_acc(sub, zeros, acc)
            plsc.subcore_barrier()

            fire_g(0, 0)

            def pipe(k2, _):
                wait_g(0)
                fire_s(2 * k2, 0, sem_s0)
                @pl.when(k2 > 0)
                def _():
                    wait_s(1, sem_s1)
                fire_g(2 * k2 + 1, 1)
                wait_g(1)
                fire_s(2 * k2 + 1, 1, sem_s1)
                wait_s(0, sem_s0)
                @pl.when(k2 < NG // 2 - 1)
                def _():
                    fire_g(2 * k2 + 2, 0)
                return 0

            lax.fori_loop(0, NG // 2, pipe, 0)
            wait_s(1, sem_s1)
            plsc.subcore_barrier()
            _copy_out(sub, acc, out, (core * P + p) * N)

        if with_deg:
            plsc.subcore_barrier()
            pltpu.sync_copy(ones, rows.at[0, pl.ds(0, SRO)])
            _zero_acc(sub, zeros, acc)
            plsc.subcore_barrier()

            def deg_body(r, _):
                pltpu.sync_copy(rows.at[0, pl.ds(0, SRO)],
                                acc.at[idx_d.at[r]], add=True)
                return 0

            lax.fori_loop(0, NSR, deg_body, 0)
            plsc.subcore_barrier()
            @pl.when(core == 0)
            def _():
                _copy_out(sub, acc, out, P * NC * N)

    return segsum


_segsum_w256_deg = _make_segsum(2, True)
_segsum_w512 = _make_segsum(4, False)
_segsum_w256 = _make_segsum(2, False)

_TC_R = 2000


def _layer_body(h_ref, agg_ref, deg_ref, ws_ref, wn_ref, b_ref, out_ref):
    inv = 1.0 / jnp.maximum(deg_ref[...], 1.0)
    mean = agg_ref[...] * inv
    acc = jnp.dot(h_ref[...], ws_ref[...], preferred_element_type=jnp.float32)
    acc = acc + jnp.dot(mean, wn_ref[...], preferred_element_type=jnp.float32)
    out_ref[...] = jnp.maximum(acc + b_ref[...], 0.0)


def _tc_layer(h, agg, deg, Ws, Wn, b):
    fin, fout = Ws.shape
    return pl.pallas_call(
        _layer_body,
        grid=(N // _TC_R,),
        in_specs=[
            pl.BlockSpec((_TC_R, fin), lambda i: (i, 0)),
            pl.BlockSpec((_TC_R, fin), lambda i: (i, 0)),
            pl.BlockSpec((_TC_R, 1), lambda i: (i, 0)),
            pl.BlockSpec((fin, fout), lambda i: (0, 0)),
            pl.BlockSpec((fin, fout), lambda i: (0, 0)),
            pl.BlockSpec((1, fout), lambda i: (0, 0)),
        ],
        out_specs=pl.BlockSpec((_TC_R, fout), lambda i: (i, 0)),
        out_shape=jax.ShapeDtypeStruct((N, fout), jnp.float32),
    )(h, agg, deg, Ws, Wn, b.reshape(1, fout))


def _proj_body(h_ref, w_ref, out_ref):
    out_ref[...] = jnp.dot(h_ref[...], w_ref[...],
                           preferred_element_type=jnp.float32)


def _tc_proj(h, W):
    fin, fout = W.shape
    return pl.pallas_call(
        _proj_body,
        grid=(N // _TC_R,),
        in_specs=[
            pl.BlockSpec((_TC_R, fin), lambda i: (i, 0)),
            pl.BlockSpec((fin, fout), lambda i: (0, 0)),
        ],
        out_specs=pl.BlockSpec((_TC_R, fout), lambda i: (i, 0)),
        out_shape=jax.ShapeDtypeStruct((N, fout), jnp.float32),
    )(h, W)


def _final_body(h_ref, agg_ref, deg_ref, ws_ref, b_ref, out_ref):
    inv = 1.0 / jnp.maximum(deg_ref[...], 1.0)
    acc = jnp.dot(h_ref[...], ws_ref[...], preferred_element_type=jnp.float32)
    out_ref[...] = jnp.maximum(acc + agg_ref[...] * inv + b_ref[...], 0.0)


def _tc_final(h, agg, deg, Ws, b):
    fin, fout = Ws.shape
    return pl.pallas_call(
        _final_body,
        grid=(N // _TC_R,),
        in_specs=[
            pl.BlockSpec((_TC_R, fin), lambda i: (i, 0)),
            pl.BlockSpec((_TC_R, fout), lambda i: (i, 0)),
            pl.BlockSpec((_TC_R, 1), lambda i: (i, 0)),
            pl.BlockSpec((fin, fout), lambda i: (0, 0)),
            pl.BlockSpec((1, fout), lambda i: (0, 0)),
        ],
        out_specs=pl.BlockSpec((_TC_R, fout), lambda i: (i, 0)),
        out_shape=jax.ShapeDtypeStruct((N, fout), jnp.float32),
    )(h, agg, deg, Ws, b.reshape(1, fout))


def _to_chunks(h, P):
    return h.reshape(N, P * NC, CH).transpose(1, 0, 2).reshape(P * NC * N, CH)


def _from_chunks(a, P):
    return a.reshape(P * NC, N, CH).transpose(1, 0, 2).reshape(N, P * NC * CH)


def kernel(x, edge_index, Ws0, Wn0, b0, Ws1, Wn1, b1, Ws2, Wn2, b2):
    src = edge_index[0].astype(jnp.int32)
    dst = edge_index[1].astype(jnp.int32)
    pad = EPAD - E
    src1 = jnp.concatenate([src, jnp.zeros((pad,), jnp.int32)])
    dst2 = jnp.concatenate([dst, jnp.full((pad,), N, jnp.int32)]).reshape(EPAD // SRO, SRO)

    zeros = jnp.zeros((TAIL_Z, CH), jnp.float32)
    ones = jnp.ones((SRO, CH), jnp.float32)

    out0 = _segsum_w256_deg(_to_chunks(x, 2), src1, dst2, zeros, ones)
    agg0 = _from_chunks(out0[: 2 * NC * N], 2)
    deg = out0[2 * NC * N :, :1]
    h1 = _tc_layer(x, agg0, deg, Ws0, Wn0, b0)

    agg1 = _from_chunks(
        _segsum_w512(_to_chunks(h1, 4), src1, dst2, zeros, ones), 4)
    h2 = _tc_layer(h1, agg1, deg, Ws1, Wn1, b1)

    hp = _tc_proj(h2, Wn2)
    agg2 = _from_chunks(
        _segsum_w256(_to_chunks(hp, 2), src1, dst2, zeros, ones), 2)
    return _tc_final(h2, agg2, deg, Ws2, b2)


# speedup vs baseline: 3.2449x; 1.0669x over previous
import functools

import jax
import jax.numpy as jnp
from jax import lax
from jax.experimental import pallas as pl
from jax.experimental.pallas import tpu as pltpu
from jax.experimental.pallas import tpu_sc as plsc

N = 10000
E = 160000
CH = 64
EPAD = 163840
EROWS = EPAD // 128
NC, NS = 2, 16
NPAD = 10016
SLAB = 624
TAIL_O = N - 15 * SLAB
TAIL_Z = NPAD - 15 * SLAB
IR = 128
NR = EPAD // IR // NS
NB = 4

_MESH = plsc.VectorSubcoreMesh(core_axis_name="c", subcore_axis_name="s")


def _zero_acc(sub, zeros, acc):
    @pl.when(sub < NS - 1)
    def _():
        pltpu.sync_copy(zeros.at[pl.ds(0, SLAB)],
                        acc.at[pl.ds(sub * SLAB, SLAB)])
    @pl.when(sub == NS - 1)
    def _():
        pltpu.sync_copy(zeros, acc.at[pl.ds(15 * SLAB, TAIL_Z)])


def _copy_out(sub, acc, out, off):
    @pl.when(sub < NS - 1)
    def _():
        pltpu.sync_copy(acc.at[pl.ds(sub * SLAB, SLAB)],
                        out.at[pl.ds(off + sub * SLAB, SLAB)])
    @pl.when(sub == NS - 1)
    def _():
        pltpu.sync_copy(acc.at[pl.ds(15 * SLAB, TAIL_O)],
                        out.at[pl.ds(off + 15 * SLAB, TAIL_O)])


def _make_segsum(P, with_deg):
    n_out = P * NC * N + (N if with_deg else 0)

    @functools.partial(
        pl.kernel,
        out_type=jax.ShapeDtypeStruct((n_out, CH), jnp.float32),
        mesh=_MESH,
        compiler_params=pltpu.CompilerParams(use_tc_tiling_on_sc=False),
        scratch_types=[
            pltpu.VMEM((NR * IR,), jnp.int32),
            pltpu.VMEM((NR, IR), jnp.int32),
            pltpu.VMEM((NB, IR, CH), jnp.float32),
            pltpu.VMEM_SHARED((NPAD, CH), jnp.float32),
            pltpu.SemaphoreType.DMA,
            [pltpu.SemaphoreType.DMA] * NB,
        ],
    )
    def segsum(h_t, src1, dst2, zeros, ones, out,
               idx_s, idx_d, rows, acc, sem_g, sem_s):
        core = lax.axis_index("c")
        sub = lax.axis_index("s")
        pltpu.sync_copy(src1.at[pl.ds(sub * NR * IR, NR * IR)], idx_s)
        pltpu.sync_copy(dst2.at[pl.ds(sub * NR, NR)], idx_d)

        def shift(delta):
            def body(i, _):
                idx_s[pl.ds(i * 16, 16)] = idx_s[pl.ds(i * 16, 16)] + delta
                return 0
            lax.fori_loop(0, NR * IR // 16, body, 0)

        NG = NR

        def fire_g(grp, buf):
            pltpu.async_copy(h_t.at[idx_s.at[pl.ds(grp * IR, IR)]],
                             rows.at[buf], sem_g)

        def wait_g(buf):
            pltpu.make_async_copy(h_t.at[idx_s.at[pl.ds(0, IR)]],
                                  rows.at[buf], sem_g).wait()

        def fire_s(grp, buf):
            pltpu.async_copy(rows.at[buf], acc.at[idx_d.at[grp]],
                             sem_s[buf], add=True)

        def wait_s(buf):
            pltpu.make_async_copy(rows.at[buf], acc.at[idx_d.at[0]],
                                  sem_s[buf]).wait()

        for p in range(P):
            shift(core * (P * N) if p == 0 else N)
            _zero_acc(sub, zeros, acc)
            plsc.subcore_barrier()

            for b in range(NB - 1):
                fire_g(b, b)

            def pipe(k, _):
                for s in range(NB):
                    t = k * NB + s
                    wait_g(s)
                    fire_s(t, s)
                    @pl.when(t > 0)
                    def _():
                        wait_s((s - 1) % NB)
                    @pl.when(t + NB - 1 < NG)
                    def _():
                        fire_g(t + NB - 1, (s - 1) % NB)
                return 0

            lax.fori_loop(0, NG // NB, pipe, 0)
            wait_s(NB - 1)
            plsc.subcore_barrier()
            _copy_out(sub, acc, out, (core * P + p) * N)

        if with_deg:
            plsc.subcore_barrier()
            pltpu.sync_copy(ones, rows.at[0])
            _zero_acc(sub, zeros, acc)
            plsc.subcore_barrier()

            def deg_body(r, _):
                pltpu.sync_copy(rows.at[0], acc.at[idx_d.at[r]], add=True)
                return 0

            lax.fori_loop(0, NR, deg_body, 0)
            plsc.subcore_barrier()
            @pl.when(core == 0)
            def _():
                _copy_out(sub, acc, out, P * NC * N)

    return segsum


_segsum_w256_deg = _make_segsum(2, True)
_segsum_w512 = _make_segsum(4, False)
_segsum_w256 = _make_segsum(2, False)

_TC_R = 2000


def _layer_body(h_ref, agg_ref, deg_ref, ws_ref, wn_ref, b_ref, out_ref):
    inv = 1.0 / jnp.maximum(deg_ref[...], 1.0)
    mean = agg_ref[...] * inv
    acc = jnp.dot(h_ref[...], ws_ref[...], preferred_element_type=jnp.float32)
    acc = acc + jnp.dot(mean, wn_ref[...], preferred_element_type=jnp.float32)
    out_ref[...] = jnp.maximum(acc + b_ref[...], 0.0)


def _tc_layer(h, agg, deg, Ws, Wn, b):
    fin, fout = Ws.shape
    return pl.pallas_call(
        _layer_body,
        grid=(N // _TC_R,),
        in_specs=[
            pl.BlockSpec((_TC_R, fin), lambda i: (i, 0)),
            pl.BlockSpec((_TC_R, fin), lambda i: (i, 0)),
            pl.BlockSpec((_TC_R, 1), lambda i: (i, 0)),
            pl.BlockSpec((fin, fout), lambda i: (0, 0)),
            pl.BlockSpec((fin, fout), lambda i: (0, 0)),
            pl.BlockSpec((1, fout), lambda i: (0, 0)),
        ],
        out_specs=pl.BlockSpec((_TC_R, fout), lambda i: (i, 0)),
        out_shape=jax.ShapeDtypeStruct((N, fout), jnp.float32),
    )(h, agg, deg, Ws, Wn, b.reshape(1, fout))


def _proj_body(h_ref, w_ref, out_ref):
    out_ref[...] = jnp.dot(h_ref[...], w_ref[...],
                           preferred_element_type=jnp.float32)


def _tc_proj(h, W):
    fin, fout = W.shape
    return pl.pallas_call(
        _proj_body,
        grid=(N // _TC_R,),
        in_specs=[
            pl.BlockSpec((_TC_R, fin), lambda i: (i, 0)),
            pl.BlockSpec((fin, fout), lambda i: (0, 0)),
        ],
        out_specs=pl.BlockSpec((_TC_R, fout), lambda i: (i, 0)),
        out_shape=jax.ShapeDtypeStruct((N, fout), jnp.float32),
    )(h, W)


def _final_body(h_ref, agg_ref, deg_ref, ws_ref, b_ref, out_ref):
    inv = 1.0 / jnp.maximum(deg_ref[...], 1.0)
    acc = jnp.dot(h_ref[...], ws_ref[...], preferred_element_type=jnp.float32)
    out_ref[...] = jnp.maximum(acc + agg_ref[...] * inv + b_ref[...], 0.0)


def _tc_final(h, agg, deg, Ws, b):
    fin, fout = Ws.shape
    return pl.pallas_call(
        _final_body,
        grid=(N // _TC_R,),
        in_specs=[
            pl.BlockSpec((_TC_R, fin), lambda i: (i, 0)),
            pl.BlockSpec((_TC_R, fout), lambda i: (i, 0)),
            pl.BlockSpec((_TC_R, 1), lambda i: (i, 0)),
            pl.BlockSpec((fin, fout), lambda i: (0, 0)),
            pl.BlockSpec((1, fout), lambda i: (0, 0)),
        ],
        out_specs=pl.BlockSpec((_TC_R, fout), lambda i: (i, 0)),
        out_shape=jax.ShapeDtypeStruct((N, fout), jnp.float32),
    )(h, agg, deg, Ws, b.reshape(1, fout))


def _to_chunks(h, P):
    return h.reshape(N, P * NC, CH).transpose(1, 0, 2).reshape(P * NC * N, CH)


def _from_chunks(a, P):
    return a.reshape(P * NC, N, CH).transpose(1, 0, 2).reshape(N, P * NC * CH)


def kernel(x, edge_index, Ws0, Wn0, b0, Ws1, Wn1, b1, Ws2, Wn2, b2):
    src = edge_index[0].astype(jnp.int32)
    dst = edge_index[1].astype(jnp.int32)
    pad = EPAD - E
    src1 = jnp.concatenate([src, jnp.zeros((pad,), jnp.int32)])
    dst2 = jnp.concatenate([dst, jnp.full((pad,), N, jnp.int32)]).reshape(EPAD // IR, IR)

    zeros = jnp.zeros((TAIL_Z, CH), jnp.float32)
    ones = jnp.ones((IR, CH), jnp.float32)

    out0 = _segsum_w256_deg(_to_chunks(x, 2), src1, dst2, zeros, ones)
    agg0 = _from_chunks(out0[: 2 * NC * N], 2)
    deg = out0[2 * NC * N :, :1]
    h1 = _tc_layer(x, agg0, deg, Ws0, Wn0, b0)

    agg1 = _from_chunks(
        _segsum_w512(_to_chunks(h1, 4), src1, dst2, zeros, ones), 4)
    h2 = _tc_layer(h1, agg1, deg, Ws1, Wn1, b1)

    hp = _tc_proj(h2, Wn2)
    agg2 = _from_chunks(
        _segsum_w256(_to_chunks(hp, 2), src1, dst2, zeros, ones), 2)
    return _tc_final(h2, agg2, deg, Ws2, b2)
